# baseline (speedup 1.0000x reference)
import jax
import jax.numpy as jnp
from jax.experimental import pallas as pl
from jax.experimental.pallas import tpu as pltpu

_BLOCK_N = 1024


def _logits_kernel(wt_ref, xt_ref, b_ref, o_ref):
    wk = jnp.concatenate([wt_ref[...], b_ref[...][None, :]], axis=0)
    xk = jnp.concatenate(
        [xt_ref[...], jnp.ones((1, xt_ref.shape[1]), dtype=jnp.float32)], axis=0
    )
    o_ref[...] = jax.lax.dot_general(
        wk,
        xk,
        (((0,), (0,)), ((), ())),
        preferred_element_type=jnp.float32,
    )


def kernel(inputs, labels, W, b):
    del labels
    B, D = inputs.shape
    N = W.shape[0]
    wt = W.T
    xt = inputs.T
    grid = (pl.cdiv(N, _BLOCK_N),)
    out_t = pl.pallas_call(
        _logits_kernel,
        grid=grid,
        in_specs=[
            pl.BlockSpec((D, _BLOCK_N), lambda i: (0, i)),
            pl.BlockSpec((D, B), lambda i: (0, 0)),
            pl.BlockSpec((_BLOCK_N,), lambda i: (i,)),
        ],
        out_specs=pl.BlockSpec((_BLOCK_N, B), lambda i: (i, 0)),
        out_shape=jax.ShapeDtypeStruct((N, B), jnp.float32),
        compiler_params=pltpu.CompilerParams(
            dimension_semantics=("arbitrary",),
        ),
    )(wt, xt, b)
    return out_t.T


# speedup vs baseline: 1.1139x; 1.1139x over previous
import jax
import jax.numpy as jnp
from jax.experimental import pallas as pl
from jax.experimental.pallas import tpu as pltpu

_BLOCK_N = 3072


def _logits_kernel(wt_ref, xt_ref, b_ref, o_ref):
    wk = jnp.concatenate([wt_ref[...], b_ref[...][None, :]], axis=0)
    xk = jnp.concatenate(
        [xt_ref[...], jnp.ones((1, xt_ref.shape[1]), dtype=jnp.float32)], axis=0
    )
    o_ref[...] = jax.lax.dot_general(
        wk,
        xk,
        (((0,), (0,)), ((), ())),
        preferred_element_type=jnp.float32,
    )


def kernel(inputs, labels, W, b):
    del labels
    B, D = inputs.shape
    N = W.shape[0]
    wt = W.T
    xt = inputs.T
    grid = (pl.cdiv(N, _BLOCK_N),)
    out_t = pl.pallas_call(
        _logits_kernel,
        grid=grid,
        in_specs=[
            pl.BlockSpec((D, _BLOCK_N), lambda i: (0, i)),
            pl.BlockSpec((D, B), lambda i: (0, 0)),
            pl.BlockSpec((_BLOCK_N,), lambda i: (i,)),
        ],
        out_specs=pl.BlockSpec((_BLOCK_N, B), lambda i: (i, 0)),
        out_shape=jax.ShapeDtypeStruct((N, B), jnp.float32),
        compiler_params=pltpu.CompilerParams(
            dimension_semantics=("arbitrary",),
        ),
    )(wt, xt, b)
    return out_t.T
